# baseline (speedup 1.0000x reference)
import functools

import jax
import jax.numpy as jnp
from jax.experimental import pallas as pl
from jax.experimental.pallas import tpu as pltpu

N = 2048
CP = 128
H = 16
BJ = 512
NL = 8
NJ = N // BJ

_HIGH = jax.lax.Precision.HIGHEST


def _gat_stack_kernel(limit_ref, swe_ref, feat_ref, ew_ref, wl_ref, bl_ref,
                      wr_ref, br_ref, invwe_ref, aw_ref, attc_ref, bias_ref,
                      out_ref, x_s, xl_s, xlp_s, xrpt_s, u_s, xnext_s):
    l = pl.program_id(0)
    jb = pl.program_id(1)

    @pl.when(jb == 0)
    def _layer_setup():
        @pl.when(l == 0)
        def _():
            x_s[...] = feat_ref[...]

        @pl.when(l > 0)
        def _():
            x_s[:, 0:H] = xnext_s[...]
            x_s[:, H:CP] = jnp.zeros((N, CP - H), jnp.float32)

        x = x_s[...]
        xl = (jnp.dot(x, wl_ref[0], precision=_HIGH,
                      preferred_element_type=jnp.float32)
              + bl_ref[0])
        xr = (jnp.dot(x, wr_ref[0], precision=_HIGH,
                      preferred_element_type=jnp.float32)
              + br_ref[0])
        xl_s[...] = xl
        xlp_s[...] = xl * invwe_ref[0]
        xrpt_s[...] = (xr * invwe_ref[0]).T
        attc = attc_ref[0]
        u_s[...] = 0.6 * jnp.dot(xl, attc, precision=_HIGH,
                                 preferred_element_type=jnp.float32)

    e = ew_ref[...]
    xl = xl_s[...]
    col0 = jb * BJ

    acc = e * swe_ref[l, 0] + u_s[...]
    for k in range(H):
        q = e + xlp_s[:, k:k + 1] + xrpt_s[k:k + 1, pl.ds(col0, BJ)]
        acc = acc + aw_ref[0, 0, k] * jnp.abs(q)

    valid = e > limit_ref[0, 0]
    masked = jnp.where(valid, acc, -1e30)
    mx = jnp.max(masked, axis=0, keepdims=True)
    ex = jnp.exp(masked - mx)
    den = jnp.sum(ex, axis=0, keepdims=True)
    num = jax.lax.dot_general(ex, xl, (((0,), (0,)), ((), ())),
                              precision=_HIGH,
                              preferred_element_type=jnp.float32)
    recip = (1.0 / (den + 1e-16)).T
    outb = num * recip + bias_ref[0]
    xnext_s[pl.ds(col0, BJ), :] = outb
    out_ref[...] = outb[None]


@functools.partial(jax.jit, static_argnames=())
def _run(features, edge_weights, limit, swe, wl, bl, wr, br, invwe, aw,
         attc, bias):
    grid = (NL, NJ)
    out = pl.pallas_call(
        _gat_stack_kernel,
        grid=grid,
        in_specs=[
            pl.BlockSpec(memory_space=pltpu.SMEM),
            pl.BlockSpec(memory_space=pltpu.SMEM),
            pl.BlockSpec((N, CP), lambda l, j: (0, 0)),
            pl.BlockSpec((N, BJ), lambda l, j: (0, j)),
            pl.BlockSpec((1, CP, H), lambda l, j: (l, 0, 0)),
            pl.BlockSpec((1, 1, H), lambda l, j: (l, 0, 0)),
            pl.BlockSpec((1, CP, H), lambda l, j: (l, 0, 0)),
            pl.BlockSpec((1, 1, H), lambda l, j: (l, 0, 0)),
            pl.BlockSpec((1, 1, H), lambda l, j: (l, 0, 0)),
            pl.BlockSpec((1, 1, H), lambda l, j: (l, 0, 0)),
            pl.BlockSpec((1, H, 1), lambda l, j: (l, 0, 0)),
            pl.BlockSpec((1, 1, H), lambda l, j: (l, 0, 0)),
        ],
        out_specs=pl.BlockSpec((1, BJ, H), lambda l, j: (l, j, 0)),
        out_shape=jax.ShapeDtypeStruct((NL, N, H), jnp.float32),
        scratch_shapes=[
            pltpu.VMEM((N, CP), jnp.float32),
            pltpu.VMEM((N, H), jnp.float32),
            pltpu.VMEM((N, H), jnp.float32),
            pltpu.VMEM((H, N), jnp.float32),
            pltpu.VMEM((N, 1), jnp.float32),
            pltpu.VMEM((N, H), jnp.float32),
        ],
        compiler_params=pltpu.CompilerParams(
            dimension_semantics=("arbitrary", "arbitrary"),
        ),
    )(limit, swe, features, edge_weights, wl, bl, wr, br, invwe, aw,
      attc, bias)
    return jnp.mean(out[NL - 1, :, :1], axis=0)


def kernel(features, edge_weights, threashold, params):
    x = jnp.squeeze(features).astype(jnp.float32)
    ew = jnp.squeeze(edge_weights).astype(jnp.float32)
    limit = (1.0 / threashold) * jnp.ones((1, 1), jnp.float32)

    feat = jnp.zeros((N, CP), jnp.float32).at[:, :x.shape[1]].set(x)

    def pad_w(w):
        return jnp.zeros((CP, H), jnp.float32).at[:w.shape[0], :w.shape[1]].set(w)

    def pad_v(v):
        v = jnp.ravel(v)
        return jnp.zeros((1, H), jnp.float32).at[0, :v.shape[0]].set(v)

    wl = jnp.stack([pad_w(p["Wl"]) for p in params])
    wr = jnp.stack([pad_w(p["Wr"]) for p in params])
    bl = jnp.stack([pad_v(p["bl"]) for p in params])
    br = jnp.stack([pad_v(p["br"]) for p in params])
    we = jnp.stack([pad_v(p["We"]) for p in params])
    att = jnp.stack([pad_v(p["att"]) for p in params])
    bias = jnp.stack([pad_v(p["bias"]) for p in params])

    invwe = jnp.where(we == 0.0, 0.0, 1.0 / we)
    aw = 0.4 * att * jnp.abs(we)
    swe = 0.6 * jnp.sum(att * we, axis=-1)
    attc = jnp.transpose(att, (0, 2, 1))

    return _run(feat, ew, limit, swe, wl, bl, wr, br, invwe, aw, attc, bias)


# speedup vs baseline: 1.0429x; 1.0429x over previous
import functools

import jax
import jax.numpy as jnp
from jax.experimental import pallas as pl
from jax.experimental.pallas import tpu as pltpu

N = 2048
CP = 128
H = 16
BJ = 512
NL = 8
NJ = N // BJ

_HIGH = jax.lax.Precision.HIGHEST


def _gat_stack_kernel(limit_ref, swe_ref, feat_ref, ew_ref, wl_ref, bl_ref,
                      wr_ref, br_ref, invwe_ref, aw_ref, attc_ref, bias_ref,
                      out_ref, x_s, xl_s, xlp_s, xrpt_s, u_s, vrow_s, xnext_s):
    l = pl.program_id(0)
    jb = pl.program_id(1)

    @pl.when(jb == 0)
    def _layer_setup():
        @pl.when(l == 0)
        def _():
            x_s[...] = feat_ref[...]

        @pl.when(l > 0)
        def _():
            x_s[:, 0:H] = xnext_s[...]
            x_s[:, H:CP] = jnp.zeros((N, CP - H), jnp.float32)

        x = x_s[...]
        xl = (jnp.dot(x, wl_ref[0],
                      preferred_element_type=jnp.float32)
              + bl_ref[0])
        xr = (jnp.dot(x, wr_ref[0],
                      preferred_element_type=jnp.float32)
              + br_ref[0])
        xl_s[...] = xl
        xlp_s[...] = xl * invwe_ref[0]
        xrpt_s[...] = (xr * invwe_ref[0]).T
        attc = attc_ref[0]
        u_s[...] = 0.6 * jnp.dot(xl, attc, precision=_HIGH,
                                 preferred_element_type=jnp.float32)
        vrow_s[...] = (0.6 * jnp.dot(xr, attc, precision=_HIGH,
                                     preferred_element_type=jnp.float32)).T

    e = ew_ref[...]
    xl = xl_s[...]
    col0 = jb * BJ

    acc = e * swe_ref[l, 0] + u_s[...] + vrow_s[0:1, pl.ds(col0, BJ)]
    for k in range(H):
        q = e + xlp_s[:, k:k + 1] + xrpt_s[k:k + 1, pl.ds(col0, BJ)]
        acc = acc + aw_ref[0, 0, k] * jnp.abs(q)

    valid = e > limit_ref[0, 0]
    masked = jnp.where(valid, acc, -1e30)
    mx = jnp.max(masked, axis=0, keepdims=True)
    ex = jnp.where(valid, jnp.exp(acc - mx), 0.0)
    den = jnp.sum(ex, axis=0, keepdims=True)
    num = jax.lax.dot_general(ex, xl, (((0,), (0,)), ((), ())),
                              precision=_HIGH,
                              preferred_element_type=jnp.float32)
    recip = (1.0 / (den + 1e-16)).T
    outb = num * recip + bias_ref[0]
    xnext_s[pl.ds(col0, BJ), :] = outb
    out_ref[...] = outb[None]


@functools.partial(jax.jit, static_argnames=())
def _run(features, edge_weights, limit, swe, wl, bl, wr, br, invwe, aw,
         attc, bias):
    grid = (NL, NJ)
    out = pl.pallas_call(
        _gat_stack_kernel,
        grid=grid,
        in_specs=[
            pl.BlockSpec(memory_space=pltpu.SMEM),
            pl.BlockSpec(memory_space=pltpu.SMEM),
            pl.BlockSpec((N, CP), lambda l, j: (0, 0)),
            pl.BlockSpec((N, BJ), lambda l, j: (0, j)),
            pl.BlockSpec((1, CP, H), lambda l, j: (l, 0, 0)),
            pl.BlockSpec((1, 1, H), lambda l, j: (l, 0, 0)),
            pl.BlockSpec((1, CP, H), lambda l, j: (l, 0, 0)),
            pl.BlockSpec((1, 1, H), lambda l, j: (l, 0, 0)),
            pl.BlockSpec((1, 1, H), lambda l, j: (l, 0, 0)),
            pl.BlockSpec((1, 1, H), lambda l, j: (l, 0, 0)),
            pl.BlockSpec((1, H, 1), lambda l, j: (l, 0, 0)),
            pl.BlockSpec((1, 1, H), lambda l, j: (l, 0, 0)),
        ],
        out_specs=pl.BlockSpec((1, BJ, H), lambda l, j: (l, j, 0)),
        out_shape=jax.ShapeDtypeStruct((NL, N, H), jnp.float32),
        scratch_shapes=[
            pltpu.VMEM((N, CP), jnp.float32),
            pltpu.VMEM((N, H), jnp.float32),
            pltpu.VMEM((N, H), jnp.float32),
            pltpu.VMEM((H, N), jnp.float32),
            pltpu.VMEM((N, 1), jnp.float32),
            pltpu.VMEM((1, N), jnp.float32),
            pltpu.VMEM((N, H), jnp.float32),
        ],
        compiler_params=pltpu.CompilerParams(
            dimension_semantics=("arbitrary", "arbitrary"),
        ),
    )(limit, swe, features, edge_weights, wl, bl, wr, br, invwe, aw,
      attc, bias)
    return jnp.mean(out[NL - 1, :, :1], axis=0)


def kernel(features, edge_weights, threashold, params):
    x = jnp.squeeze(features).astype(jnp.float32)
    ew = jnp.squeeze(edge_weights).astype(jnp.float32)
    limit = (1.0 / threashold) * jnp.ones((1, 1), jnp.float32)

    feat = jnp.zeros((N, CP), jnp.float32).at[:, :x.shape[1]].set(x)

    def pad_w(w):
        return jnp.zeros((CP, H), jnp.float32).at[:w.shape[0], :w.shape[1]].set(w)

    def pad_v(v):
        v = jnp.ravel(v)
        return jnp.zeros((1, H), jnp.float32).at[0, :v.shape[0]].set(v)

    wl = jnp.stack([pad_w(p["Wl"]) for p in params])
    wr = jnp.stack([pad_w(p["Wr"]) for p in params])
    bl = jnp.stack([pad_v(p["bl"]) for p in params])
    br = jnp.stack([pad_v(p["br"]) for p in params])
    we = jnp.stack([pad_v(p["We"]) for p in params])
    att = jnp.stack([pad_v(p["att"]) for p in params])
    bias = jnp.stack([pad_v(p["bias"]) for p in params])

    invwe = jnp.where(we == 0.0, 0.0, 1.0 / we)
    aw = 0.4 * att * jnp.abs(we)
    swe = 0.6 * jnp.sum(att * we, axis=-1)
    attc = jnp.transpose(att, (0, 2, 1))

    return _run(feat, ew, limit, swe, wl, bl, wr, br, invwe, aw, attc, bias)
